# baseline (speedup 1.0000x reference)
import functools

import jax
import jax.numpy as jnp
import numpy as np
from jax import lax
from jax.experimental import pallas as pl
from jax.experimental.pallas import tpu as pltpu
from jax.experimental.pallas import tpu_sc as plsc

B = 4096
ND = 13
NS = 26
V = 100000
E = 128
NF = NS + 1
P = 32
H0 = 1024

NW = 32
ROWS = B * NS
RPW = ROWS // NW
CHUNK = 128
NCHUNK = RPW // CHUNK

NCHUNK = B * P // (NW * CHUNK)


@functools.cache
def _make_sc_gather():
    mesh = plsc.VectorSubcoreMesh(core_axis_name="c", subcore_axis_name="s")

    @functools.partial(
        pl.kernel,
        mesh=mesh,
        out_type=jax.ShapeDtypeStruct((B * P, E), jnp.float32),
        scratch_types=[
            pltpu.VMEM((NCHUNK, CHUNK), jnp.int32),
            pltpu.VMEM((CHUNK, E), jnp.float32),
            pltpu.VMEM((CHUNK, E), jnp.float32),
            pltpu.SemaphoreType.DMA,
            pltpu.SemaphoreType.DMA,
        ],
    )
    def _sc_gather(emb_hbm, idx_hbm, out_hbm, idx_v, rows_a, rows_b, sem_a, sem_b):
        wid = lax.axis_index("s") * 2 + lax.axis_index("c")
        base = wid * (NCHUNK * CHUNK)
        pltpu.sync_copy(idx_hbm.at[wid], idx_v)

        def body(p, carry):
            c0 = 2 * p
            ha = pltpu.async_copy(emb_hbm.at[idx_v.at[c0]], rows_a, sem_a)
            hb = pltpu.async_copy(emb_hbm.at[idx_v.at[c0 + 1]], rows_b, sem_b)
            ha.wait()
            pltpu.sync_copy(rows_a, out_hbm.at[pl.ds(base + c0 * CHUNK, CHUNK)])
            hb.wait()
            pltpu.sync_copy(rows_b, out_hbm.at[pl.ds(base + (c0 + 1) * CHUNK, CHUNK)])
            return carry

        lax.fori_loop(0, NCHUNK // 2, body, 0)

    return _sc_gather


BB = 256
_IU0, _IU1 = np.triu_indices(NF)
NPAIR = _IU0.shape[0]
NPAD = 384

_SLOT = np.concatenate([[P - 1], np.arange(NS)])
_S0, _S1 = _SLOT[_IU0], _SLOT[_IU1]
_SEL_NP = np.zeros((P * P, NPAD), np.float32)
_SEL_NP[_S0 * P + _S1, np.arange(NPAIR)] += 0.5
_SEL_NP[_S1 * P + _S0, np.arange(NPAIR)] += 0.5


def _tc_body(x_ref, eb_ref, wb0, bb0, wb1, bb1, wb2, bb2,
             w0a, sel, w0p, bt0, wt1, bt1, wt2, bt2, wt3, bt3, wt4, bt4, o_ref):
    f32 = jnp.float32
    dense = x_ref[:, :ND]
    h = jnp.maximum(jnp.dot(dense, wb0[:], preferred_element_type=f32) + bb0[:], 0.0)
    h = jnp.maximum(jnp.dot(h, wb1[:], preferred_element_type=f32) + bb1[:], 0.0)
    bot = jnp.maximum(jnp.dot(h, wb2[:], preferred_element_type=f32) + bb2[:], 0.0)
    l = lax.broadcasted_iota(jnp.int32, (BB, P, E), 1)
    feats = jnp.where(l == P - 1, bot[:, None, :],
                      jnp.where(l <= NS - 1, eb_ref[:], 0.0))
    gram = lax.dot_general(feats, feats, (((2,), (2,)), ((0,), (0,))),
                           preferred_element_type=f32)
    gflat = gram.reshape(BB, P * P)
    acts = jnp.dot(gflat, sel[:], preferred_element_type=f32)
    h = jnp.dot(bot, w0a[:], preferred_element_type=f32)
    h = h + jnp.dot(acts, w0p[:], preferred_element_type=f32)
    h = jnp.maximum(h + bt0[:], 0.0)
    h = jnp.maximum(jnp.dot(h, wt1[:], preferred_element_type=f32) + bt1[:], 0.0)
    h = jnp.maximum(jnp.dot(h, wt2[:], preferred_element_type=f32) + bt2[:], 0.0)
    h = jnp.maximum(jnp.dot(h, wt3[:], preferred_element_type=f32) + bt3[:], 0.0)
    o_ref[:, :] = jnp.dot(h, wt4[:], preferred_element_type=f32) + bt4[:]


def _const_spec(shape):
    nd = len(shape)
    return pl.BlockSpec(shape, lambda i: (0,) * nd)


def _tc_forward(x, embed, wb0, bb0, wb1, bb1, wb2, bb2,
                w0a, sel, w0p, bt0, wt1, bt1, wt2, bt2, wt3, bt3, wt4, bt4):
    nblk = B // BB
    consts = [wb0, bb0, wb1, bb1, wb2, bb2, w0a, sel, w0p, bt0,
              wt1, bt1, wt2, bt2, wt3, bt3, wt4, bt4]
    in_specs = [
        pl.BlockSpec((BB, ND + NS), lambda i: (i, 0)),
        pl.BlockSpec((BB, P, E), lambda i: (i, 0, 0)),
    ] + [_const_spec(c.shape) for c in consts]
    return pl.pallas_call(
        _tc_body,
        grid=(nblk,),
        in_specs=in_specs,
        out_specs=pl.BlockSpec((BB, 1), lambda i: (i, 0)),
        out_shape=jax.ShapeDtypeStruct((B, 1), jnp.float32),
        compiler_params=pltpu.CompilerParams(
            dimension_semantics=("arbitrary",)),
    )(x, embed, *consts)


def kernel(x, emb, Wb0, bb0, Wb1, bb1, Wb2, bb2,
           Wt0, bt0, Wt1, bt1, Wt2, bt2, Wt3, bt3, Wt4, bt4):
    cat = x[:, ND:].astype(jnp.int32)
    offs = (jnp.arange(NS, dtype=jnp.int32) * V)[None, :]
    idx = jnp.concatenate(
        [cat + offs, jnp.zeros((B, P - NS), jnp.int32)], axis=1
    ).reshape(NW, NCHUNK, CHUNK)

    sel = jnp.asarray(_SEL_NP)
    w0p = jnp.concatenate(
        [Wt0[E:], jnp.zeros((NPAD - NPAIR, H0), jnp.float32)], axis=0)
    w0a = Wt0[:E]

    embed = _make_sc_gather()(emb, idx).reshape(B, P, E)

    out = _tc_forward(
        x, embed, Wb0, bb0.reshape(1, -1), Wb1, bb1.reshape(1, -1),
        Wb2, bb2.reshape(1, -1), w0a, sel, w0p, bt0.reshape(1, -1),
        Wt1, bt1.reshape(1, -1), Wt2, bt2.reshape(1, -1),
        Wt3, bt3.reshape(1, -1), Wt4, bt4.reshape(1, -1))
    return out


# speedup vs baseline: 7.8790x; 7.8790x over previous
import functools

import jax
import jax.numpy as jnp
import numpy as np
from jax import lax
from jax.experimental import pallas as pl
from jax.experimental.pallas import tpu as pltpu
from jax.experimental.pallas import tpu_sc as plsc

B = 4096
ND = 13
NS = 26
V = 100000
E = 128
NF = NS + 1
P = 32
H0 = 1024

NW = 32
ROWS = B * NS
RPW = ROWS // NW
CHUNK = 128
NCHUNK = RPW // CHUNK

NCHUNK = B * P // (NW * CHUNK)


@functools.cache
def _make_sc_gather():
    mesh = plsc.VectorSubcoreMesh(core_axis_name="c", subcore_axis_name="s")

    @functools.partial(
        pl.kernel,
        mesh=mesh,
        out_type=jax.ShapeDtypeStruct((B * P, E), jnp.float32),
        scratch_types=[
            pltpu.VMEM((NCHUNK, CHUNK), jnp.int32),
            pltpu.VMEM((CHUNK, E), jnp.float32),
            pltpu.VMEM((CHUNK, E), jnp.float32),
            pltpu.SemaphoreType.DMA,
            pltpu.SemaphoreType.DMA,
        ],
    )
    def _sc_gather(emb_hbm, idx_hbm, out_hbm, idx_v, rows_a, rows_b, sem_a, sem_b):
        wid = lax.axis_index("s") * 2 + lax.axis_index("c")
        base = wid * (NCHUNK * CHUNK)
        pltpu.sync_copy(idx_hbm.at[wid], idx_v)

        def body(p, carry):
            c0 = 2 * p
            ha = pltpu.async_copy(emb_hbm.at[idx_v.at[c0]], rows_a, sem_a)
            hb = pltpu.async_copy(emb_hbm.at[idx_v.at[c0 + 1]], rows_b, sem_b)
            ha.wait()
            pltpu.sync_copy(rows_a, out_hbm.at[pl.ds(base + c0 * CHUNK, CHUNK)])
            hb.wait()
            pltpu.sync_copy(rows_b, out_hbm.at[pl.ds(base + (c0 + 1) * CHUNK, CHUNK)])
            return carry

        lax.fori_loop(0, NCHUNK // 2, body, 0)

    return _sc_gather


BB = 256
_IU0, _IU1 = np.triu_indices(NF)
NPAIR = _IU0.shape[0]
NPAD = 384

_SLOT = np.concatenate([[P - 1], np.arange(NS)])
_S0, _S1 = _SLOT[_IU0], _SLOT[_IU1]
_SEL_NP = np.zeros((P * P, NPAD), np.float32)
_SEL_NP[_S0 * P + _S1, np.arange(NPAIR)] += 0.5
_SEL_NP[_S1 * P + _S0, np.arange(NPAIR)] += 0.5


def _tc_body(x_ref, eb_ref, wb0, bb0, wb1, bb1, wb2, bb2,
             w0a, sel, w0p, bt0, wt1, bt1, wt2, bt2, wt3, bt3, wt4, bt4, o_ref):
    f32 = jnp.float32
    dense = x_ref[:, :ND]
    h = jnp.maximum(jnp.dot(dense, wb0[:], preferred_element_type=f32) + bb0[:], 0.0)
    h = jnp.maximum(jnp.dot(h, wb1[:], preferred_element_type=f32) + bb1[:], 0.0)
    bot = jnp.maximum(jnp.dot(h, wb2[:], preferred_element_type=f32) + bb2[:], 0.0)
    l = lax.broadcasted_iota(jnp.int32, (BB, P, E), 1)
    feats = jnp.where(l == P - 1, bot[:, None, :],
                      jnp.where(l <= NS - 1, eb_ref[:], 0.0))
    gram = lax.dot_general(feats, feats, (((2,), (2,)), ((0,), (0,))),
                           preferred_element_type=f32)
    gflat = gram.reshape(BB, P * P)
    acts = jnp.dot(gflat, sel[:], preferred_element_type=f32)
    h = jnp.dot(bot, w0a[:], preferred_element_type=f32)
    h = h + jnp.dot(acts, w0p[:], preferred_element_type=f32)
    h = jnp.maximum(h + bt0[:], 0.0)
    h = jnp.maximum(jnp.dot(h, wt1[:], preferred_element_type=f32) + bt1[:], 0.0)
    h = jnp.maximum(jnp.dot(h, wt2[:], preferred_element_type=f32) + bt2[:], 0.0)
    h = jnp.maximum(jnp.dot(h, wt3[:], preferred_element_type=f32) + bt3[:], 0.0)
    o_ref[:, :] = jnp.dot(h, wt4[:], preferred_element_type=f32) + bt4[:]


def _const_spec(shape):
    nd = len(shape)
    return pl.BlockSpec(shape, lambda i: (0,) * nd)


def _tc_forward(x, embed, wb0, bb0, wb1, bb1, wb2, bb2,
                w0a, sel, w0p, bt0, wt1, bt1, wt2, bt2, wt3, bt3, wt4, bt4):
    nblk = B // BB
    consts = [wb0, bb0, wb1, bb1, wb2, bb2, w0a, sel, w0p, bt0,
              wt1, bt1, wt2, bt2, wt3, bt3, wt4, bt4]
    in_specs = [
        pl.BlockSpec((BB, ND + NS), lambda i: (i, 0)),
        pl.BlockSpec((BB, P, E), lambda i: (i, 0, 0)),
    ] + [_const_spec(c.shape) for c in consts]
    return pl.pallas_call(
        _tc_body,
        grid=(nblk,),
        in_specs=in_specs,
        out_specs=pl.BlockSpec((BB, 1), lambda i: (i, 0)),
        out_shape=jax.ShapeDtypeStruct((B, 1), jnp.float32),
        compiler_params=pltpu.CompilerParams(
            dimension_semantics=("arbitrary",)),
    )(x, embed, *consts)


def kernel(x, emb, Wb0, bb0, Wb1, bb1, Wb2, bb2,
           Wt0, bt0, Wt1, bt1, Wt2, bt2, Wt3, bt3, Wt4, bt4):
    cat = x[:, ND:].astype(jnp.int32)
    offs = (jnp.arange(NS, dtype=jnp.int32) * V)[None, :]
    dummy = (jnp.arange(B * (P - NS), dtype=jnp.int32) % (NS * V)).reshape(
        B, P - NS)
    idx = jnp.concatenate([cat + offs, dummy], axis=1).reshape(
        NW, NCHUNK, CHUNK)

    sel = jnp.asarray(_SEL_NP)
    w0p = jnp.concatenate(
        [Wt0[E:], jnp.zeros((NPAD - NPAIR, H0), jnp.float32)], axis=0)
    w0a = Wt0[:E]

    embed = _make_sc_gather()(emb, idx).reshape(B, P, E)

    out = _tc_forward(
        x, embed, Wb0, bb0.reshape(1, -1), Wb1, bb1.reshape(1, -1),
        Wb2, bb2.reshape(1, -1), w0a, sel, w0p, bt0.reshape(1, -1),
        Wt1, bt1.reshape(1, -1), Wt2, bt2.reshape(1, -1),
        Wt3, bt3.reshape(1, -1), Wt4, bt4.reshape(1, -1))
    return out


# speedup vs baseline: 8.8150x; 1.1188x over previous
import functools

import jax
import jax.numpy as jnp
import numpy as np
from jax import lax
from jax.experimental import pallas as pl
from jax.experimental.pallas import tpu as pltpu
from jax.experimental.pallas import tpu_sc as plsc

B = 4096
ND = 13
NS = 26
V = 100000
E = 128
NF = NS + 1
P = 32
H0 = 1024

NW = 32
ROWS = B * NS
RPW = ROWS // NW
CHUNK = 128
NCHUNK = RPW // CHUNK

NSPLIT = 2
NB = B // NSPLIT
NCHUNK = NB * P // (NW * CHUNK)


@functools.cache
def _make_sc_gather():
    mesh = plsc.VectorSubcoreMesh(core_axis_name="c", subcore_axis_name="s")

    @functools.partial(
        pl.kernel,
        mesh=mesh,
        out_type=jax.ShapeDtypeStruct((NB * P, E), jnp.float32),
        scratch_types=[
            pltpu.VMEM((NCHUNK, CHUNK), jnp.int32),
            pltpu.VMEM((CHUNK, E), jnp.float32),
            pltpu.VMEM((CHUNK, E), jnp.float32),
            pltpu.SemaphoreType.DMA,
            pltpu.SemaphoreType.DMA,
        ],
    )
    def _sc_gather(emb_hbm, idx_hbm, out_hbm, idx_v, rows_a, rows_b, sem_a, sem_b):
        wid = lax.axis_index("s") * 2 + lax.axis_index("c")
        base = wid * (NCHUNK * CHUNK)
        pltpu.sync_copy(idx_hbm.at[wid], idx_v)

        def body(p, carry):
            c0 = 2 * p
            ha = pltpu.async_copy(emb_hbm.at[idx_v.at[c0]], rows_a, sem_a)
            hb = pltpu.async_copy(emb_hbm.at[idx_v.at[c0 + 1]], rows_b, sem_b)
            ha.wait()
            pltpu.sync_copy(rows_a, out_hbm.at[pl.ds(base + c0 * CHUNK, CHUNK)])
            hb.wait()
            pltpu.sync_copy(rows_b, out_hbm.at[pl.ds(base + (c0 + 1) * CHUNK, CHUNK)])
            return carry

        lax.fori_loop(0, NCHUNK // 2, body, 0)

    return _sc_gather


BB = 256
_IU0, _IU1 = np.triu_indices(NF)
NPAIR = _IU0.shape[0]
NPAD = 384

_SLOT = np.concatenate([[P - 1], np.arange(NS)])
_S0, _S1 = _SLOT[_IU0], _SLOT[_IU1]
_SEL_NP = np.zeros((P * P, NPAD), np.float32)
_SEL_NP[_S0 * P + _S1, np.arange(NPAIR)] += 0.5
_SEL_NP[_S1 * P + _S0, np.arange(NPAIR)] += 0.5


def _tc_body(x_ref, eb_ref, wb0, bb0, wb1, bb1, wb2, bb2,
             w0a, sel, w0p, bt0, wt1, bt1, wt2, bt2, wt3, bt3, wt4, bt4, o_ref):
    f32 = jnp.float32
    dense = x_ref[:, :ND]
    h = jnp.maximum(jnp.dot(dense, wb0[:], preferred_element_type=f32) + bb0[:], 0.0)
    h = jnp.maximum(jnp.dot(h, wb1[:], preferred_element_type=f32) + bb1[:], 0.0)
    bot = jnp.maximum(jnp.dot(h, wb2[:], preferred_element_type=f32) + bb2[:], 0.0)
    l = lax.broadcasted_iota(jnp.int32, (BB, P, E), 1)
    feats = jnp.where(l == P - 1, bot[:, None, :],
                      jnp.where(l <= NS - 1, eb_ref[:], 0.0))
    gram = lax.dot_general(feats, feats, (((2,), (2,)), ((0,), (0,))),
                           preferred_element_type=f32)
    gflat = gram.reshape(BB, P * P)
    acts = jnp.dot(gflat, sel[:], preferred_element_type=f32)
    h = jnp.dot(bot, w0a[:], preferred_element_type=f32)
    h = h + jnp.dot(acts, w0p[:], preferred_element_type=f32)
    h = jnp.maximum(h + bt0[:], 0.0)
    h = jnp.maximum(jnp.dot(h, wt1[:], preferred_element_type=f32) + bt1[:], 0.0)
    h = jnp.maximum(jnp.dot(h, wt2[:], preferred_element_type=f32) + bt2[:], 0.0)
    h = jnp.maximum(jnp.dot(h, wt3[:], preferred_element_type=f32) + bt3[:], 0.0)
    o_ref[:, :] = jnp.dot(h, wt4[:], preferred_element_type=f32) + bt4[:]


def _const_spec(shape):
    nd = len(shape)
    return pl.BlockSpec(shape, lambda i: (0,) * nd)


def _tc_forward(x, embed, wb0, bb0, wb1, bb1, wb2, bb2,
                w0a, sel, w0p, bt0, wt1, bt1, wt2, bt2, wt3, bt3, wt4, bt4):
    nblk = NB // BB
    consts = [wb0, bb0, wb1, bb1, wb2, bb2, w0a, sel, w0p, bt0,
              wt1, bt1, wt2, bt2, wt3, bt3, wt4, bt4]
    in_specs = [
        pl.BlockSpec((BB, ND + NS), lambda i: (i, 0)),
        pl.BlockSpec((BB, P, E), lambda i: (i, 0, 0)),
    ] + [_const_spec(c.shape) for c in consts]
    return pl.pallas_call(
        _tc_body,
        grid=(nblk,),
        in_specs=in_specs,
        out_specs=pl.BlockSpec((BB, 1), lambda i: (i, 0)),
        out_shape=jax.ShapeDtypeStruct((NB, 1), jnp.float32),
        compiler_params=pltpu.CompilerParams(
            dimension_semantics=("arbitrary",)),
    )(x, embed, *consts)


def kernel(x, emb, Wb0, bb0, Wb1, bb1, Wb2, bb2,
           Wt0, bt0, Wt1, bt1, Wt2, bt2, Wt3, bt3, Wt4, bt4):
    cat = x[:, ND:].astype(jnp.int32)
    offs = (jnp.arange(NS, dtype=jnp.int32) * V)[None, :]
    dummy = (jnp.arange(B * (P - NS), dtype=jnp.int32) % (NS * V)).reshape(
        B, P - NS)
    idx = jnp.concatenate([cat + offs, dummy], axis=1).reshape(
        NSPLIT, NW, NCHUNK, CHUNK)

    sel = jnp.asarray(_SEL_NP)
    w0p = jnp.concatenate(
        [Wt0[E:], jnp.zeros((NPAD - NPAIR, H0), jnp.float32)], axis=0)
    w0a = Wt0[:E]

    gather = _make_sc_gather()
    outs = []
    for k in range(NSPLIT):
        embed = gather(emb, idx[k]).reshape(NB, P, E)
        outs.append(_tc_forward(
            x[k * NB:(k + 1) * NB], embed,
            Wb0, bb0.reshape(1, -1), Wb1, bb1.reshape(1, -1),
            Wb2, bb2.reshape(1, -1), w0a, sel, w0p, bt0.reshape(1, -1),
            Wt1, bt1.reshape(1, -1), Wt2, bt2.reshape(1, -1),
            Wt3, bt3.reshape(1, -1), Wt4, bt4.reshape(1, -1)))
    return jnp.concatenate(outs, axis=0)


# speedup vs baseline: 9.3197x; 1.0573x over previous
import functools

import jax
import jax.numpy as jnp
import numpy as np
from jax import lax
from jax.experimental import pallas as pl
from jax.experimental.pallas import tpu as pltpu
from jax.experimental.pallas import tpu_sc as plsc

B = 4096
ND = 13
NS = 26
V = 100000
E = 128
NF = NS + 1
P = 32
H0 = 1024

NW = 32
ROWS = B * NS
RPW = ROWS // NW
CHUNK = 128
NCHUNK = RPW // CHUNK

NSPLIT = 2
NB = B // NSPLIT
NCHUNK = NB * P // (NW * CHUNK)


@functools.cache
def _make_sc_gather():
    mesh = plsc.VectorSubcoreMesh(core_axis_name="c", subcore_axis_name="s")

    @functools.partial(
        pl.kernel,
        mesh=mesh,
        out_type=jax.ShapeDtypeStruct((NB * P, E), jnp.float32),
        scratch_types=[
            pltpu.VMEM((NCHUNK, CHUNK), jnp.int32),
            pltpu.VMEM((CHUNK, E), jnp.float32),
            pltpu.VMEM((CHUNK, E), jnp.float32),
            pltpu.SemaphoreType.DMA,
            pltpu.SemaphoreType.DMA,
        ],
    )
    def _sc_gather(emb_hbm, idx_hbm, out_hbm, idx_v, rows_a, rows_b, sem_a, sem_b):
        wid = lax.axis_index("s") * 2 + lax.axis_index("c")
        base = wid * (NCHUNK * CHUNK)
        pltpu.sync_copy(idx_hbm.at[wid], idx_v)

        def body(p, carry):
            c0 = 2 * p
            ha = pltpu.async_copy(emb_hbm.at[idx_v.at[c0]], rows_a, sem_a)
            hb = pltpu.async_copy(emb_hbm.at[idx_v.at[c0 + 1]], rows_b, sem_b)
            ha.wait()
            pltpu.sync_copy(rows_a, out_hbm.at[pl.ds(base + c0 * CHUNK, CHUNK)])
            hb.wait()
            pltpu.sync_copy(rows_b, out_hbm.at[pl.ds(base + (c0 + 1) * CHUNK, CHUNK)])
            return carry

        lax.fori_loop(0, NCHUNK // 2, body, 0)

    return _sc_gather


BB = 512
_IU0, _IU1 = np.triu_indices(NF)
NPAIR = _IU0.shape[0]
NPAD = 384

_SLOT = np.concatenate([[P - 1], np.arange(NS)])
_S0, _S1 = _SLOT[_IU0], _SLOT[_IU1]
_SEL_NP = np.zeros((P * P, NPAD), np.float32)
_SEL_NP[_S0 * P + _S1, np.arange(NPAIR)] += 0.5
_SEL_NP[_S1 * P + _S0, np.arange(NPAIR)] += 0.5


def _bot_body(x_ref, wb0, bb0, wb1, bb1, wb2, bb2, o_ref):
    f32 = jnp.float32
    dense = x_ref[:, :ND]
    h = jnp.maximum(jnp.dot(dense, wb0[:], preferred_element_type=f32) + bb0[:], 0.0)
    h = jnp.maximum(jnp.dot(h, wb1[:], preferred_element_type=f32) + bb1[:], 0.0)
    o_ref[:, :] = jnp.maximum(
        jnp.dot(h, wb2[:], preferred_element_type=f32) + bb2[:], 0.0)


def _bot_forward(x, wb0, bb0, wb1, bb1, wb2, bb2):
    nblk = B // BB
    consts = [wb0, bb0, wb1, bb1, wb2, bb2]
    in_specs = [pl.BlockSpec((BB, ND + NS), lambda i: (i, 0))] + [
        _const_spec(c.shape) for c in consts]
    return pl.pallas_call(
        _bot_body,
        grid=(nblk,),
        in_specs=in_specs,
        out_specs=pl.BlockSpec((BB, E), lambda i: (i, 0)),
        out_shape=jax.ShapeDtypeStruct((B, E), jnp.float32),
        compiler_params=pltpu.CompilerParams(
            dimension_semantics=("arbitrary",)),
    )(x, *consts)


def _tc_body(bot_ref, eb_ref,
             w0a, sel, w0p, bt0, wt1, bt1, wt2, bt2, wt3, bt3, wt4, bt4, o_ref):
    f32 = jnp.float32
    bot = bot_ref[:]
    l = lax.broadcasted_iota(jnp.int32, (BB, P, E), 1)
    feats = jnp.where(l == P - 1, bot[:, None, :],
                      jnp.where(l <= NS - 1, eb_ref[:], 0.0))
    gram = lax.dot_general(feats, feats, (((2,), (2,)), ((0,), (0,))),
                           preferred_element_type=f32)
    gflat = gram.reshape(BB, P * P)
    acts = jnp.dot(gflat, sel[:], preferred_element_type=f32)
    h = jnp.dot(bot, w0a[:], preferred_element_type=f32)
    h = h + jnp.dot(acts, w0p[:], preferred_element_type=f32)
    h = jnp.maximum(h + bt0[:], 0.0)
    h = jnp.maximum(jnp.dot(h, wt1[:], preferred_element_type=f32) + bt1[:], 0.0)
    h = jnp.maximum(jnp.dot(h, wt2[:], preferred_element_type=f32) + bt2[:], 0.0)
    h = jnp.maximum(jnp.dot(h, wt3[:], preferred_element_type=f32) + bt3[:], 0.0)
    o_ref[:, :] = jnp.dot(h, wt4[:], preferred_element_type=f32) + bt4[:]


def _const_spec(shape):
    nd = len(shape)
    return pl.BlockSpec(shape, lambda i: (0,) * nd)


def _tc_forward(bot, embed, split,
                w0a, sel, w0p, bt0, wt1, bt1, wt2, bt2, wt3, bt3, wt4, bt4):
    nblk = NB // BB
    boff = split * nblk
    consts = [w0a, sel, w0p, bt0,
              wt1, bt1, wt2, bt2, wt3, bt3, wt4, bt4]
    in_specs = [
        pl.BlockSpec((BB, E), lambda i: (i + boff, 0)),
        pl.BlockSpec((BB, P, E), lambda i: (i, 0, 0)),
    ] + [_const_spec(c.shape) for c in consts]
    return pl.pallas_call(
        _tc_body,
        grid=(nblk,),
        in_specs=in_specs,
        out_specs=pl.BlockSpec((BB, 1), lambda i: (i, 0)),
        out_shape=jax.ShapeDtypeStruct((NB, 1), jnp.float32),
        compiler_params=pltpu.CompilerParams(
            dimension_semantics=("arbitrary",)),
    )(bot, embed, *consts)


def kernel(x, emb, Wb0, bb0, Wb1, bb1, Wb2, bb2,
           Wt0, bt0, Wt1, bt1, Wt2, bt2, Wt3, bt3, Wt4, bt4):
    cat = x[:, ND:].astype(jnp.int32)
    offs = (jnp.arange(NS, dtype=jnp.int32) * V)[None, :]
    dummy = (jnp.arange(B * (P - NS), dtype=jnp.int32) % (NS * V)).reshape(
        B, P - NS)
    idx = jnp.concatenate([cat + offs, dummy], axis=1).reshape(
        NSPLIT, NW, NCHUNK, CHUNK)

    sel = jnp.asarray(_SEL_NP)
    w0p = jnp.concatenate(
        [Wt0[E:], jnp.zeros((NPAD - NPAIR, H0), jnp.float32)], axis=0)
    w0a = Wt0[:E]

    gather = _make_sc_gather()
    bot = _bot_forward(x, Wb0, bb0.reshape(1, -1), Wb1, bb1.reshape(1, -1),
                       Wb2, bb2.reshape(1, -1))
    outs = []
    for k in range(NSPLIT):
        embed = gather(emb, idx[k]).reshape(NB, P, E)
        outs.append(_tc_forward(
            bot, embed, k,
            w0a, sel, w0p, bt0.reshape(1, -1),
            Wt1, bt1.reshape(1, -1), Wt2, bt2.reshape(1, -1),
            Wt3, bt3.reshape(1, -1), Wt4, bt4.reshape(1, -1)))
    return jnp.concatenate(outs, axis=0)


# speedup vs baseline: 9.5184x; 1.0213x over previous
import functools

import jax
import jax.numpy as jnp
import numpy as np
from jax import lax
from jax.experimental import pallas as pl
from jax.experimental.pallas import tpu as pltpu
from jax.experimental.pallas import tpu_sc as plsc

B = 4096
ND = 13
NS = 26
V = 100000
E = 128
NF = NS + 1
P = 32
H0 = 1024

NW = 32
ROWS = B * NS
RPW = ROWS // NW
CHUNK = 128
NCHUNK = RPW // CHUNK

NSPLIT = 2
NB = B // NSPLIT
SPW = NB // NW


@functools.cache
def _make_sc_gather():
    mesh = plsc.VectorSubcoreMesh(core_axis_name="c", subcore_axis_name="s")

    @functools.partial(
        pl.kernel,
        mesh=mesh,
        out_type=jax.ShapeDtypeStruct((NB, P, E), jnp.float32),
        scratch_types=[
            pltpu.VMEM((NS, SPW), jnp.int32),
            pltpu.VMEM((SPW, E), jnp.float32),
            pltpu.VMEM((SPW, E), jnp.float32),
            pltpu.SemaphoreType.DMA,
            pltpu.SemaphoreType.DMA,
        ],
    )
    def _sc_gather(emb_hbm, idx_hbm, out_hbm, idx_v, rows_a, rows_b, sem_a, sem_b):
        wid = lax.axis_index("s") * 2 + lax.axis_index("c")
        b0 = wid * SPW
        pltpu.sync_copy(idx_hbm.at[wid], idx_v)

        def body(p, carry):
            s0 = 2 * p
            ha = pltpu.async_copy(emb_hbm.at[idx_v.at[s0]], rows_a, sem_a)
            hb = pltpu.async_copy(emb_hbm.at[idx_v.at[s0 + 1]], rows_b, sem_b)
            ha.wait()
            pltpu.sync_copy(rows_a, out_hbm.at[pl.ds(b0, SPW), s0])
            hb.wait()
            pltpu.sync_copy(rows_b, out_hbm.at[pl.ds(b0, SPW), s0 + 1])
            return carry

        lax.fori_loop(0, NS // 2, body, 0)

    return _sc_gather


BB = 512
_IU0, _IU1 = np.triu_indices(NF)
NPAIR = _IU0.shape[0]
NPAD = 384

_SLOT = np.concatenate([[P - 1], np.arange(NS)])
_S0, _S1 = _SLOT[_IU0], _SLOT[_IU1]
_SEL_NP = np.zeros((P * P, NPAD), np.float32)
_SEL_NP[_S0 * P + _S1, np.arange(NPAIR)] += 0.5
_SEL_NP[_S1 * P + _S0, np.arange(NPAIR)] += 0.5


def _bot_body(x_ref, wb0, bb0, wb1, bb1, wb2, bb2, o_ref):
    f32 = jnp.float32
    dense = x_ref[:, :ND]
    h = jnp.maximum(jnp.dot(dense, wb0[:], preferred_element_type=f32) + bb0[:], 0.0)
    h = jnp.maximum(jnp.dot(h, wb1[:], preferred_element_type=f32) + bb1[:], 0.0)
    o_ref[:, :] = jnp.maximum(
        jnp.dot(h, wb2[:], preferred_element_type=f32) + bb2[:], 0.0)


def _bot_forward(x, wb0, bb0, wb1, bb1, wb2, bb2):
    nblk = B // BB
    consts = [wb0, bb0, wb1, bb1, wb2, bb2]
    in_specs = [pl.BlockSpec((BB, ND + NS), lambda i: (i, 0))] + [
        _const_spec(c.shape) for c in consts]
    return pl.pallas_call(
        _bot_body,
        grid=(nblk,),
        in_specs=in_specs,
        out_specs=pl.BlockSpec((BB, E), lambda i: (i, 0)),
        out_shape=jax.ShapeDtypeStruct((B, E), jnp.float32),
        compiler_params=pltpu.CompilerParams(
            dimension_semantics=("arbitrary",)),
    )(x, *consts)


def _tc_body(bot_ref, eb_ref,
             w0a, sel, w0p, bt0, wt1, bt1, wt2, bt2, wt3, bt3, wt4, bt4, o_ref):
    f32 = jnp.float32
    bot = bot_ref[:]
    l = lax.broadcasted_iota(jnp.int32, (BB, P, E), 1)
    feats = jnp.where(l == P - 1, bot[:, None, :],
                      jnp.where(l <= NS - 1, eb_ref[:], 0.0))
    gram = lax.dot_general(feats, feats, (((2,), (2,)), ((0,), (0,))),
                           preferred_element_type=f32)
    gflat = gram.reshape(BB, P * P)
    acts = jnp.dot(gflat, sel[:], preferred_element_type=f32)
    h = jnp.dot(bot, w0a[:], preferred_element_type=f32)
    h = h + jnp.dot(acts, w0p[:], preferred_element_type=f32)
    h = jnp.maximum(h + bt0[:], 0.0)
    h = jnp.maximum(jnp.dot(h, wt1[:], preferred_element_type=f32) + bt1[:], 0.0)
    h = jnp.maximum(jnp.dot(h, wt2[:], preferred_element_type=f32) + bt2[:], 0.0)
    h = jnp.maximum(jnp.dot(h, wt3[:], preferred_element_type=f32) + bt3[:], 0.0)
    o_ref[:, :] = jnp.dot(h, wt4[:], preferred_element_type=f32) + bt4[:]


def _const_spec(shape):
    nd = len(shape)
    return pl.BlockSpec(shape, lambda i: (0,) * nd)


def _tc_forward(bot, embed, split,
                w0a, sel, w0p, bt0, wt1, bt1, wt2, bt2, wt3, bt3, wt4, bt4):
    nblk = NB // BB
    boff = split * nblk
    consts = [w0a, sel, w0p, bt0,
              wt1, bt1, wt2, bt2, wt3, bt3, wt4, bt4]
    in_specs = [
        pl.BlockSpec((BB, E), lambda i: (i + boff, 0)),
        pl.BlockSpec((BB, P, E), lambda i: (i, 0, 0)),
    ] + [_const_spec(c.shape) for c in consts]
    return pl.pallas_call(
        _tc_body,
        grid=(nblk,),
        in_specs=in_specs,
        out_specs=pl.BlockSpec((BB, 1), lambda i: (i, 0)),
        out_shape=jax.ShapeDtypeStruct((NB, 1), jnp.float32),
        compiler_params=pltpu.CompilerParams(
            dimension_semantics=("arbitrary",)),
    )(bot, embed, *consts)


def kernel(x, emb, Wb0, bb0, Wb1, bb1, Wb2, bb2,
           Wt0, bt0, Wt1, bt1, Wt2, bt2, Wt3, bt3, Wt4, bt4):
    cat = x[:, ND:].astype(jnp.int32)
    offs = (jnp.arange(NS, dtype=jnp.int32) * V)[None, None, :]
    idx = (cat.reshape(NSPLIT, NW, SPW, NS) + offs).transpose(0, 1, 3, 2)

    sel = jnp.asarray(_SEL_NP)
    w0p = jnp.concatenate(
        [Wt0[E:], jnp.zeros((NPAD - NPAIR, H0), jnp.float32)], axis=0)
    w0a = Wt0[:E]

    gather = _make_sc_gather()
    bot = _bot_forward(x, Wb0, bb0.reshape(1, -1), Wb1, bb1.reshape(1, -1),
                       Wb2, bb2.reshape(1, -1))
    outs = []
    for k in range(NSPLIT):
        embed = gather(emb, idx[k])
        outs.append(_tc_forward(
            bot, embed, k,
            w0a, sel, w0p, bt0.reshape(1, -1),
            Wt1, bt1.reshape(1, -1), Wt2, bt2.reshape(1, -1),
            Wt3, bt3.reshape(1, -1), Wt4, bt4.reshape(1, -1)))
    return jnp.concatenate(outs, axis=0)


# speedup vs baseline: 9.9125x; 1.0414x over previous
import functools

import jax
import jax.numpy as jnp
import numpy as np
from jax import lax
from jax.experimental import pallas as pl
from jax.experimental.pallas import tpu as pltpu
from jax.experimental.pallas import tpu_sc as plsc

B = 4096
ND = 13
NS = 26
V = 100000
E = 128
NF = NS + 1
P = 32
H0 = 1024

NW = 32
ROWS = B * NS
RPW = ROWS // NW
CHUNK = 128
NCHUNK = RPW // CHUNK

NSPLIT = 2
NB = B // NSPLIT
SPW = NB // NW


@functools.cache
def _make_sc_gather():
    mesh = plsc.VectorSubcoreMesh(core_axis_name="c", subcore_axis_name="s")

    @functools.partial(
        pl.kernel,
        mesh=mesh,
        out_type=jax.ShapeDtypeStruct((NB, P, E), jnp.float32),
        scratch_types=[
            pltpu.VMEM((NS, SPW), jnp.int32),
            pltpu.VMEM((SPW, E), jnp.float32),
            pltpu.VMEM((SPW, E), jnp.float32),
            pltpu.VMEM((SPW, E), jnp.float32),
            pltpu.VMEM((SPW, E), jnp.float32),
            pltpu.SemaphoreType.DMA,
            pltpu.SemaphoreType.DMA,
            pltpu.SemaphoreType.DMA,
            pltpu.SemaphoreType.DMA,
            pltpu.SemaphoreType.DMA,
            pltpu.SemaphoreType.DMA,
            pltpu.SemaphoreType.DMA,
            pltpu.SemaphoreType.DMA,
        ],
    )
    def _sc_gather(emb_hbm, idx_hbm, out_hbm, idx_v,
                   ra, rb, rc, rd, ga, gb, gc, gd, wa, wb, wc, wd):
        wid = lax.axis_index("s") * 2 + lax.axis_index("c")
        b0 = wid * SPW
        pltpu.sync_copy(idx_hbm.at[wid], idx_v)

        def gath(s, buf, sem):
            return pltpu.async_copy(emb_hbm.at[idx_v.at[s]], buf, sem)

        def wr(s, buf, sem):
            return pltpu.async_copy(buf, out_hbm.at[pl.ds(b0, SPW), s], sem)

        def body(q, carry):
            s0 = 4 * q
            ha = gath(s0, ra, ga)
            hb = gath(s0 + 1, rb, gb)
            ha.wait()
            ka = wr(s0, ra, wa)
            hc = gath(s0 + 2, rc, gc)
            hb.wait()
            kb = wr(s0 + 1, rb, wb)
            hd = gath(s0 + 3, rd, gd)
            hc.wait()
            kc = wr(s0 + 2, rc, wc)
            hd.wait()
            kd = wr(s0 + 3, rd, wd)
            ka.wait()
            kb.wait()
            kc.wait()
            kd.wait()
            return carry

        lax.fori_loop(0, NS // 4, body, 0)
        ha = gath(NS - 2, ra, ga)
        hb = gath(NS - 1, rb, gb)
        ha.wait()
        ka = wr(NS - 2, ra, wa)
        hb.wait()
        kb = wr(NS - 1, rb, wb)
        ka.wait()
        kb.wait()

    return _sc_gather


BB = 512
_IU0, _IU1 = np.triu_indices(NF)
NPAIR = _IU0.shape[0]
NPAD = 384

_SLOT = np.concatenate([[P - 1], np.arange(NS)])
_S0, _S1 = _SLOT[_IU0], _SLOT[_IU1]
_SEL_NP = np.zeros((P * P, NPAD), np.float32)
_SEL_NP[_S0 * P + _S1, np.arange(NPAIR)] += 0.5
_SEL_NP[_S1 * P + _S0, np.arange(NPAIR)] += 0.5


def _bot_body(x_ref, wb0, bb0, wb1, bb1, wb2, bb2, o_ref):
    f32 = jnp.float32
    dense = x_ref[:, :ND]
    h = jnp.maximum(jnp.dot(dense, wb0[:], preferred_element_type=f32) + bb0[:], 0.0)
    h = jnp.maximum(jnp.dot(h, wb1[:], preferred_element_type=f32) + bb1[:], 0.0)
    o_ref[:, :] = jnp.maximum(
        jnp.dot(h, wb2[:], preferred_element_type=f32) + bb2[:], 0.0)


def _bot_forward(x, wb0, bb0, wb1, bb1, wb2, bb2):
    nblk = B // BB
    consts = [wb0, bb0, wb1, bb1, wb2, bb2]
    in_specs = [pl.BlockSpec((BB, ND + NS), lambda i: (i, 0))] + [
        _const_spec(c.shape) for c in consts]
    return pl.pallas_call(
        _bot_body,
        grid=(nblk,),
        in_specs=in_specs,
        out_specs=pl.BlockSpec((BB, E), lambda i: (i, 0)),
        out_shape=jax.ShapeDtypeStruct((B, E), jnp.float32),
        compiler_params=pltpu.CompilerParams(
            dimension_semantics=("arbitrary",)),
    )(x, *consts)


def _tc_body(bot_ref, eb_ref,
             w0a, sel, w0p, bt0, wt1, bt1, wt2, bt2, wt3, bt3, wt4, bt4, o_ref):
    f32 = jnp.float32
    bot = bot_ref[:]
    l = lax.broadcasted_iota(jnp.int32, (BB, P, E), 1)
    feats = jnp.where(l == P - 1, bot[:, None, :],
                      jnp.where(l <= NS - 1, eb_ref[:], 0.0))
    gram = lax.dot_general(feats, feats, (((2,), (2,)), ((0,), (0,))),
                           preferred_element_type=f32)
    gflat = gram.reshape(BB, P * P)
    acts = jnp.dot(gflat, sel[:], preferred_element_type=f32)
    h = jnp.dot(bot, w0a[:], preferred_element_type=f32)
    h = h + jnp.dot(acts, w0p[:], preferred_element_type=f32)
    h = jnp.maximum(h + bt0[:], 0.0)
    h = jnp.maximum(jnp.dot(h, wt1[:], preferred_element_type=f32) + bt1[:], 0.0)
    h = jnp.maximum(jnp.dot(h, wt2[:], preferred_element_type=f32) + bt2[:], 0.0)
    h = jnp.maximum(jnp.dot(h, wt3[:], preferred_element_type=f32) + bt3[:], 0.0)
    o_ref[:, :] = jnp.dot(h, wt4[:], preferred_element_type=f32) + bt4[:]


def _const_spec(shape):
    nd = len(shape)
    return pl.BlockSpec(shape, lambda i: (0,) * nd)


def _tc_forward(bot, embed, split,
                w0a, sel, w0p, bt0, wt1, bt1, wt2, bt2, wt3, bt3, wt4, bt4):
    nblk = NB // BB
    boff = split * nblk
    consts = [w0a, sel, w0p, bt0,
              wt1, bt1, wt2, bt2, wt3, bt3, wt4, bt4]
    in_specs = [
        pl.BlockSpec((BB, E), lambda i: (i + boff, 0)),
        pl.BlockSpec((BB, P, E), lambda i: (i, 0, 0)),
    ] + [_const_spec(c.shape) for c in consts]
    return pl.pallas_call(
        _tc_body,
        grid=(nblk,),
        in_specs=in_specs,
        out_specs=pl.BlockSpec((BB, 1), lambda i: (i, 0)),
        out_shape=jax.ShapeDtypeStruct((NB, 1), jnp.float32),
        compiler_params=pltpu.CompilerParams(
            dimension_semantics=("arbitrary",)),
    )(bot, embed, *consts)


def kernel(x, emb, Wb0, bb0, Wb1, bb1, Wb2, bb2,
           Wt0, bt0, Wt1, bt1, Wt2, bt2, Wt3, bt3, Wt4, bt4):
    cat = x[:, ND:].astype(jnp.int32)
    offs = (jnp.arange(NS, dtype=jnp.int32) * V)[None, None, :]
    idx = (cat.reshape(NSPLIT, NW, SPW, NS) + offs).transpose(0, 1, 3, 2)

    sel = jnp.asarray(_SEL_NP)
    w0p = jnp.concatenate(
        [Wt0[E:], jnp.zeros((NPAD - NPAIR, H0), jnp.float32)], axis=0)
    w0a = Wt0[:E]

    gather = _make_sc_gather()
    bot = _bot_forward(x, Wb0, bb0.reshape(1, -1), Wb1, bb1.reshape(1, -1),
                       Wb2, bb2.reshape(1, -1))
    outs = []
    for k in range(NSPLIT):
        embed = gather(emb, idx[k])
        outs.append(_tc_forward(
            bot, embed, k,
            w0a, sel, w0p, bt0.reshape(1, -1),
            Wt1, bt1.reshape(1, -1), Wt2, bt2.reshape(1, -1),
            Wt3, bt3.reshape(1, -1), Wt4, bt4.reshape(1, -1)))
    return jnp.concatenate(outs, axis=0)


# speedup vs baseline: 9.9343x; 1.0022x over previous
import functools

import jax
import jax.numpy as jnp
import numpy as np
from jax import lax
from jax.experimental import pallas as pl
from jax.experimental.pallas import tpu as pltpu
from jax.experimental.pallas import tpu_sc as plsc

B = 4096
ND = 13
NS = 26
V = 100000
E = 128
NF = NS + 1
P = 32
H0 = 1024

NW = 32
NSPLIT = 2
NB = B // NSPLIT
SPW = NB // NW


@functools.cache
def _make_sc_gather():
    mesh = plsc.VectorSubcoreMesh(core_axis_name="c", subcore_axis_name="s")

    @functools.partial(
        pl.kernel,
        mesh=mesh,
        out_type=jax.ShapeDtypeStruct((NB, P, E), jnp.float32),
        scratch_types=[
            pltpu.VMEM((NS, SPW), jnp.int32),
            pltpu.VMEM((SPW, E), jnp.float32),
            pltpu.VMEM((SPW, E), jnp.float32),
            pltpu.VMEM((SPW, E), jnp.float32),
            pltpu.VMEM((SPW, E), jnp.float32),
            pltpu.SemaphoreType.DMA,
            pltpu.SemaphoreType.DMA,
            pltpu.SemaphoreType.DMA,
            pltpu.SemaphoreType.DMA,
            pltpu.SemaphoreType.DMA,
            pltpu.SemaphoreType.DMA,
            pltpu.SemaphoreType.DMA,
            pltpu.SemaphoreType.DMA,
        ],
    )
    def _sc_gather(emb_hbm, idx_hbm, out_hbm, idx_v,
                   ra, rb, rc, rd, ga, gb, gc, gd, wa, wb, wc, wd):
        wid = lax.axis_index("s") * 2 + lax.axis_index("c")
        b0 = wid * SPW
        pltpu.sync_copy(idx_hbm.at[wid], idx_v)

        def gath(s, buf, sem):
            return pltpu.async_copy(emb_hbm.at[idx_v.at[s]], buf, sem)

        def wr(s, buf, sem):
            return pltpu.async_copy(buf, out_hbm.at[pl.ds(b0, SPW), s], sem)

        def body(q, carry):
            s0 = 4 * q
            ha = gath(s0, ra, ga)
            hb = gath(s0 + 1, rb, gb)
            ha.wait()
            ka = wr(s0, ra, wa)
            hc = gath(s0 + 2, rc, gc)
            hb.wait()
            kb = wr(s0 + 1, rb, wb)
            hd = gath(s0 + 3, rd, gd)
            hc.wait()
            kc = wr(s0 + 2, rc, wc)
            hd.wait()
            kd = wr(s0 + 3, rd, wd)
            ka.wait()
            kb.wait()
            kc.wait()
            kd.wait()
            return carry

        lax.fori_loop(0, NS // 4, body, 0)
        ha = gath(NS - 2, ra, ga)
        hb = gath(NS - 1, rb, gb)
        ha.wait()
        ka = wr(NS - 2, ra, wa)
        hb.wait()
        kb = wr(NS - 1, rb, wb)
        ka.wait()
        kb.wait()

    return _sc_gather


BB = 512
_IU0, _IU1 = np.triu_indices(NF)
NPAIR = _IU0.shape[0]
NPAD = 384

_SLOT = np.concatenate([[P - 1], np.arange(NS)])
_S0, _S1 = _SLOT[_IU0], _SLOT[_IU1]
_SEL_NP = np.zeros((P * P, NPAD), np.float32)
_SEL_NP[_S0 * P + _S1, np.arange(NPAIR)] += 0.5
_SEL_NP[_S1 * P + _S0, np.arange(NPAIR)] += 0.5


def _bot_body(x_ref, wb0, bb0, wb1, bb1, wb2, bb2, o_ref):
    f32 = jnp.float32
    dense = x_ref[:, :ND]
    h = jnp.maximum(jnp.dot(dense, wb0[:], preferred_element_type=f32) + bb0[:], 0.0)
    h = jnp.maximum(jnp.dot(h, wb1[:], preferred_element_type=f32) + bb1[:], 0.0)
    o_ref[:, :] = jnp.maximum(
        jnp.dot(h, wb2[:], preferred_element_type=f32) + bb2[:], 0.0)


def _bot_forward(x, wb0, bb0, wb1, bb1, wb2, bb2):
    nblk = B // BB
    consts = [wb0, bb0, wb1, bb1, wb2, bb2]
    in_specs = [pl.BlockSpec((BB, ND + NS), lambda i: (i, 0))] + [
        _const_spec(c.shape) for c in consts]
    return pl.pallas_call(
        _bot_body,
        grid=(nblk,),
        in_specs=in_specs,
        out_specs=pl.BlockSpec((BB, E), lambda i: (i, 0)),
        out_shape=jax.ShapeDtypeStruct((B, E), jnp.float32),
        compiler_params=pltpu.CompilerParams(
            dimension_semantics=("arbitrary",)),
    )(x, *consts)


def _tc_body(bot_ref, eb_ref,
             w0a, sel, w0p, bt0, wt1, bt1, wt2, bt2, wt3, bt3, wt4, bt4, o_ref):
    f32 = jnp.float32
    bot = bot_ref[:]
    l = lax.broadcasted_iota(jnp.int32, (BB, P, E), 1)
    feats = jnp.where(l == P - 1, bot[:, None, :],
                      jnp.where(l <= NS - 1, eb_ref[:], 0.0))
    gram = lax.dot_general(feats, feats, (((2,), (2,)), ((0,), (0,))),
                           preferred_element_type=f32)
    gflat = gram.reshape(BB, P * P)
    acts = jnp.dot(gflat, sel[:], preferred_element_type=f32)
    h = jnp.dot(bot, w0a[:], preferred_element_type=f32)
    h = h + jnp.dot(acts, w0p[:], preferred_element_type=f32)
    h = jnp.maximum(h + bt0[:], 0.0)
    h = jnp.maximum(jnp.dot(h, wt1[:], preferred_element_type=f32) + bt1[:], 0.0)
    h = jnp.maximum(jnp.dot(h, wt2[:], preferred_element_type=f32) + bt2[:], 0.0)
    h = jnp.maximum(jnp.dot(h, wt3[:], preferred_element_type=f32) + bt3[:], 0.0)
    o_ref[:, :] = jnp.dot(h, wt4[:], preferred_element_type=f32) + bt4[:]


def _const_spec(shape):
    nd = len(shape)
    return pl.BlockSpec(shape, lambda i: (0,) * nd)


def _tc_forward(bot, embed, split,
                w0a, sel, w0p, bt0, wt1, bt1, wt2, bt2, wt3, bt3, wt4, bt4):
    nblk = NB // BB
    boff = split * nblk
    consts = [w0a, sel, w0p, bt0,
              wt1, bt1, wt2, bt2, wt3, bt3, wt4, bt4]
    in_specs = [
        pl.BlockSpec((BB, E), lambda i: (i + boff, 0)),
        pl.BlockSpec((BB, P, E), lambda i: (i, 0, 0)),
    ] + [_const_spec(c.shape) for c in consts]
    return pl.pallas_call(
        _tc_body,
        grid=(nblk,),
        in_specs=in_specs,
        out_specs=pl.BlockSpec((BB, 1), lambda i: (i, 0)),
        out_shape=jax.ShapeDtypeStruct((NB, 1), jnp.float32),
        compiler_params=pltpu.CompilerParams(
            dimension_semantics=("arbitrary",)),
    )(bot, embed, *consts)


def kernel(x, emb, Wb0, bb0, Wb1, bb1, Wb2, bb2,
           Wt0, bt0, Wt1, bt1, Wt2, bt2, Wt3, bt3, Wt4, bt4):
    cat = x[:, ND:].astype(jnp.int32)
    offs = (jnp.arange(NS, dtype=jnp.int32) * V)[None, None, :]
    idx = (cat.reshape(NSPLIT, NW, SPW, NS) + offs).transpose(0, 1, 3, 2)

    sel = jnp.asarray(_SEL_NP)
    w0p = jnp.concatenate(
        [Wt0[E:], jnp.zeros((NPAD - NPAIR, H0), jnp.float32)], axis=0)
    w0a = Wt0[:E]

    gather = _make_sc_gather()
    bot = _bot_forward(x, Wb0, bb0.reshape(1, -1), Wb1, bb1.reshape(1, -1),
                       Wb2, bb2.reshape(1, -1))
    outs = []
    for k in range(NSPLIT):
        embed = gather(emb, idx[k])
        outs.append(_tc_forward(
            bot, embed, k,
            w0a, sel, w0p, bt0.reshape(1, -1),
            Wt1, bt1.reshape(1, -1), Wt2, bt2.reshape(1, -1),
            Wt3, bt3.reshape(1, -1), Wt4, bt4.reshape(1, -1)))
    return jnp.concatenate(outs, axis=0)
